# baseline (speedup 1.0000x reference)
import jax
import jax.numpy as jnp
from jax import lax
from jax.experimental import pallas as pl
from jax.experimental.pallas import tpu as pltpu
from jax.experimental.pallas import tpu_sc as plsc

NBINS = 1024
EDIM = 64
_B, _N, _F = 64, 4096, 64
CH = 128
NCHUNK = _N // CH
NW = 32
BPW = _B // NW

_TROWS = 1152
_PI = 3.14159265358979
_BIN_SCALE = NBINS / (2.0 * _PI)
_ATAN_COEFFS = (-1.0 / 15.0, 1.0 / 13.0, -1.0 / 11.0, 1.0 / 9.0,
                -1.0 / 7.0, 1.0 / 5.0, -1.0 / 3.0, 1.0)


def _iota16():
    return lax.broadcasted_iota(jnp.int32, (16,), 0)


def _perm(v, idx):
    return v.at[idx].get(mode="promise_in_bounds")


def _atan2_16(y, x):
    ax, ay = jnp.abs(x), jnp.abs(y)
    hi = jnp.maximum(ax, ay)
    lo = jnp.minimum(ax, ay)
    z = lo / hi
    red = z > 0.4142135
    zr = jnp.where(red, (z - 1.0) / (z + 1.0), z)
    w = zr * zr
    p = jnp.full((16,), _ATAN_COEFFS[0], jnp.float32)
    for c in _ATAN_COEFFS[1:]:
        p = p * w + c
    at = p * zr
    at = jnp.where(red, at + (_PI / 4.0), at)
    at = jnp.where(ay > ax, (_PI / 2.0) - at, at)
    at = jnp.where(x < 0.0, _PI - at, at)
    at = jnp.where(y < 0.0, -at, at)
    return at


def _bins_from_packed(w):
    iot = _iota16()
    cx = (w & 0xFFFF).astype(jnp.float32)
    cy = lax.shift_right_logical(w, 16).astype(jnp.float32)
    sx = cx + _perm(cx, iot ^ 1)
    sx = sx + _perm(sx, iot ^ 2)
    sy = cy + _perm(cy, iot ^ 1)
    sy = sy + _perm(sy, iot ^ 2)
    rx = cx - sx * 0.25
    ry = cy - sy * 0.25
    th = _atan2_16(ry, rx)
    s = (th + _PI) * _BIN_SCALE
    bi = jnp.clip(s.astype(jnp.int32), 0, NBINS - 1)
    return jnp.where((rx == 0.0) & (ry == 0.0), NBINS, bi)


def _sc_body(x_hbm, pidx_hbm, tab0_hbm, tab1_hbm, tab2_hbm, eff_hbm, out_hbm,
             tab0_sh, tab1_sh, tab2_sh,
             b0_v, b1_v, b2_v, rows0_v, rows1_v, rows2_v, xbuf_v, eff_v,
             sem_g, sem_x, sem_o):
    wid = lax.axis_index("s") * 2 + lax.axis_index("c")
    sid = lax.axis_index("s")
    iot = _iota16()

    rpt = _TROWS // 16
    sl = pl.ds(sid * rpt, rpt)
    pltpu.sync_copy(tab0_hbm.at[sl], tab0_sh.at[sl])
    pltpu.sync_copy(tab1_hbm.at[sl], tab1_sh.at[sl])
    pltpu.sync_copy(tab2_hbm.at[sl], tab2_sh.at[sl])
    plsc.subcore_barrier()

    pltpu.sync_copy(eff_hbm, eff_v)
    a0, b0 = eff_v[0], eff_v[1]
    a1, b1 = eff_v[2], eff_v[3]
    a2, b2 = eff_v[4], eff_v[5]

    perms = [jnp.clip((iot - 4 * m) * 4, 0, 15) for m in range(4)]
    masks = [(iot >= 4 * m) & (iot < 4 * (m + 1)) for m in range(4)]

    def coarsen16(loads):
        out = None
        for m in range(4):
            t = _perm(loads[m], perms[m])
            out = t if out is None else jnp.where(masks[m], t, out)
        return out

    def group_e(load, avec, bvec):
        g = [[jnp.exp(load(t, d)) for d in range(4)] for t in range(4)]
        e = []
        for d in range(4):
            ssum = g[0][d] + g[1][d] + g[2][d] + g[3][d]
            bos = bvec / ssum
            e.append([g[t][d] * bos + avec for t in range(4)])
        return e

    def x_copy(c, b, q, inbound):
        base = b * (_N // 2) + c * (CH // 2)
        hch = CH // 2
        if inbound:
            return pltpu.make_async_copy(
                x_hbm.at[pl.ds(base, hch)],
                xbuf_v.at[pl.ds(q * hch, hch)], sem_x)
        return pltpu.make_async_copy(
            xbuf_v.at[pl.ds(q * hch, hch)],
            out_hbm.at[pl.ds(base, hch)], sem_o)

    def gathers(c, q, which=(0, 1, 2)):
        mk = [
            lambda: pltpu.make_async_copy(tab0_sh.at[b0_v.at[c]],
                                          rows0_v.at[pl.ds(q * CH, CH)],
                                          sem_g),
            lambda: pltpu.make_async_copy(tab1_sh.at[b1_v.at[c]],
                                          rows1_v.at[pl.ds(q * 32, 32)],
                                          sem_g),
            lambda: pltpu.make_async_copy(
                tab2_sh.at[b2_v.at[c // 2, pl.ds((c % 2) * 8, 8)]],
                rows2_v.at[pl.ds(q * 8, 8)], sem_g),
        ]
        return [mk[i]() for i in which]

    def batch_body(bi, carry):
        b = wid * BPW + bi
        pltpu.sync_copy(pidx_hbm.at[b], b0_v)
        x_copy(0, b, 0, True).start()
        x_copy(1, b, 1, True).start()

        def c1body(blk, c_):
            loads = [b0_v[blk // 2, pl.ds((blk % 2) * 64 + m * 16, 16)]
                     for m in range(4)]
            b1_v[blk // 2, pl.ds((blk % 2) * 16, 16)] = coarsen16(loads)
            return c_
        lax.fori_loop(0, 64, c1body, 0)

        def c2body(blk, c_):
            loads = [b1_v[blk * 2 + m // 2, pl.ds((m % 2) * 16, 16)]
                     for m in range(4)]
            b2_v[blk, pl.ds(0, 16)] = coarsen16(loads)
            return c_
        lax.fori_loop(0, 16, c2body, 0)

        def bn2(it, c_):
            b2_v[it, pl.ds(0, 16)] = _bins_from_packed(b2_v[it, pl.ds(0, 16)])
            return c_
        lax.fori_loop(0, 16, bn2, 0)

        def bn1(it, c_):
            sl = (it // 2, pl.ds((it % 2) * 16, 16))
            b1_v[sl] = _bins_from_packed(b1_v[sl])
            return c_
        lax.fori_loop(0, 64, bn1, 0)
        for d in gathers(0, 0, which=(1, 2)):
            d.start()

        def bn0(it, c_):
            sl = (it // 8, pl.ds((it % 8) * 16, 16))
            b0_v[sl] = _bins_from_packed(b0_v[sl])
            return c_
        lax.fori_loop(0, 256, bn0, 0)

        gathers(0, 0, which=(0,))[0].start()

        def chunk_body(c, c_):
            q = c % 2

            @pl.when(c + 1 < NCHUNK)
            def _():
                for d in gathers(c + 1, 1 - q):
                    d.start()

            @pl.when(c > 0)
            def _():
                x_copy(c - 1, b, 1 - q, False).wait()

            @pl.when((c > 0) & (c + 1 < NCHUNK))
            def _():
                x_copy(c + 1, b, 1 - q, True).start()

            for d in gathers(c, q):
                d.wait()
            x_copy(c, b, q, True).wait()

            def l2blk(kk2, cc_):
                e2 = group_e(
                    lambda t, d: rows2_v[q * 8 + 4 * kk2 + t,
                                         pl.ds(16 * d, 16)], a2, b2)
                for t2 in range(4):
                    kk1 = 4 * kk2 + t2
                    e1 = group_e(
                        lambda t, d: rows1_v[q * 32 + 4 * kk1 + t,
                                             pl.ds(16 * d, 16)], a1, b1)
                    pr = [[e1[d][t1] * e2[d][t2] for t1 in range(4)]
                          for d in range(4)]
                    for t1 in range(4):
                        kk = 4 * kk1 + t1
                        e0 = group_e(
                            lambda t, d: rows0_v[q * CH + 4 * kk + t,
                                                 pl.ds(16 * d, 16)], a0, b0)
                        for t0 in range(4):
                            pos = 4 * kk + t0
                            row = q * (CH // 2) + pos // 2
                            half = (pos % 2) * 64
                            for d in range(4):
                                cs = pl.ds(half + 16 * d, 16)
                                xv = xbuf_v[row, cs]
                                xbuf_v[row, cs] = \
                                    e0[d][t0] * pr[d][t1] * xv
                return cc_
            lax.fori_loop(0, 2, l2blk, 0)

            x_copy(c, b, q, False).start()
            return c_
        lax.fori_loop(0, NCHUNK, chunk_body, 0)
        x_copy(NCHUNK - 1, b, (NCHUNK - 1) % 2, False).wait()
        return carry
    lax.fori_loop(0, BPW, batch_body, 0)


def _pad_rows(tab):
    return jnp.pad(tab, ((0, _TROWS - tab.shape[0]), (0, 64)))


def kernel(x_level, indices_layer, seq_level, table_0, table_1, table_2):
    B, N, F = x_level.shape
    assert (B, N, F) == (_B, _N, _F)
    mk = (jnp.arange(3) < seq_level).astype(jnp.float32)
    scal = jnp.stack([1.0 - mk[0], mk[0], 1.0 - mk[1], mk[1],
                      1.0 - mk[2], mk[2]])
    eff = scal[:, None] * jnp.ones((1, 16), jnp.float32)

    pidx = (indices_layer[..., 0] |
            (indices_layer[..., 1] << 16)).reshape(B, NCHUNK, CH)

    mesh = plsc.VectorSubcoreMesh(core_axis_name="c", subcore_axis_name="s")
    run = pl.kernel(
        _sc_body, mesh=mesh,
        out_type=jax.ShapeDtypeStruct((B * N // 2, 2 * F), jnp.float32),
        scratch_types=[
            pltpu.VMEM_SHARED((_TROWS, 128), jnp.float32),
            pltpu.VMEM_SHARED((_TROWS, 128), jnp.float32),
            pltpu.VMEM_SHARED((_TROWS, 128), jnp.float32),
            pltpu.VMEM((NCHUNK, CH), jnp.int32),
            pltpu.VMEM((32, 32), jnp.int32),
            pltpu.VMEM((16, 16), jnp.int32),
            pltpu.VMEM((2 * CH, 128), jnp.float32),
            pltpu.VMEM((2 * 32, 128), jnp.float32),
            pltpu.VMEM((2 * 8, 128), jnp.float32),
            pltpu.VMEM((CH, 128), jnp.float32),
            pltpu.VMEM((6, 16), jnp.float32),
            pltpu.SemaphoreType.DMA,
            pltpu.SemaphoreType.DMA,
            pltpu.SemaphoreType.DMA,
        ])
    out = run(x_level.reshape(B * N // 2, 2 * F), pidx,
              _pad_rows(table_0), _pad_rows(table_1), _pad_rows(table_2), eff)
    return out.reshape(B, N, F)


# speedup vs baseline: 1.7433x; 1.7433x over previous
import jax
import jax.numpy as jnp
from jax import lax
from jax.experimental import pallas as pl
from jax.experimental.pallas import tpu as pltpu
from jax.experimental.pallas import tpu_sc as plsc

NBINS = 1024
EDIM = 64
_B, _N, _F = 64, 4096, 64
CH = 128
NCHUNK = _N // CH
NW = 32
BPW = _B // NW

_TROWS = 1152
_PI = 3.14159265358979
_BIN_SCALE = NBINS / (2.0 * _PI)
_ATAN_COEFFS = (-1.0 / 15.0, 1.0 / 13.0, -1.0 / 11.0, 1.0 / 9.0,
                -1.0 / 7.0, 1.0 / 5.0, -1.0 / 3.0, 1.0)


def _iota16():
    return lax.broadcasted_iota(jnp.int32, (16,), 0)


def _perm(v, idx):
    return v.at[idx].get(mode="promise_in_bounds")


def _atan2_16(y, x):
    ax, ay = jnp.abs(x), jnp.abs(y)
    hi = jnp.maximum(ax, ay)
    lo = jnp.minimum(ax, ay)
    z = lo / hi
    red = z > 0.4142135
    zr = jnp.where(red, (z - 1.0) / (z + 1.0), z)
    w = zr * zr
    p = jnp.full((16,), _ATAN_COEFFS[0], jnp.float32)
    for c in _ATAN_COEFFS[1:]:
        p = p * w + c
    at = p * zr
    at = jnp.where(red, at + (_PI / 4.0), at)
    at = jnp.where(ay > ax, (_PI / 2.0) - at, at)
    at = jnp.where(x < 0.0, _PI - at, at)
    at = jnp.where(y < 0.0, -at, at)
    return at


def _bins_from_packed(w):
    iot = _iota16()
    cx = (w & 0xFFFF).astype(jnp.float32)
    cy = lax.shift_right_logical(w, 16).astype(jnp.float32)
    sx = cx + _perm(cx, iot ^ 1)
    sx = sx + _perm(sx, iot ^ 2)
    sy = cy + _perm(cy, iot ^ 1)
    sy = sy + _perm(sy, iot ^ 2)
    rx = cx - sx * 0.25
    ry = cy - sy * 0.25
    th = _atan2_16(ry, rx)
    s = (th + _PI) * _BIN_SCALE
    bi = jnp.clip(s.astype(jnp.int32), 0, NBINS - 1)
    return jnp.where((rx == 0.0) & (ry == 0.0), NBINS, bi)


def _sc_body(x_hbm, pidx_hbm, tab0_hbm, tab1_hbm, tab2_hbm, eff_hbm, out_hbm,
             tab0_sh, tab1_sh, tab2_sh,
             b0_v, b1_v, b2_v, rows0_v, rows1_v, rows2_v, xbuf_v, eff_v,
             sem_g, sem_x, sem_o):
    wid = lax.axis_index("s") * 2 + lax.axis_index("c")
    sid = lax.axis_index("s")
    iot = _iota16()

    rpt = _TROWS // 16
    sl = pl.ds(sid * rpt, rpt)
    pltpu.sync_copy(tab0_hbm.at[sl], tab0_sh.at[sl])
    pltpu.sync_copy(tab1_hbm.at[sl], tab1_sh.at[sl])
    pltpu.sync_copy(tab2_hbm.at[sl], tab2_sh.at[sl])
    plsc.subcore_barrier()

    pltpu.sync_copy(eff_hbm, eff_v)
    a0, b0 = eff_v[0], eff_v[1]
    a1, b1 = eff_v[2], eff_v[3]
    a2, b2 = eff_v[4], eff_v[5]

    perms = [jnp.clip((iot - 4 * m) * 4, 0, 15) for m in range(4)]
    masks = [(iot >= 4 * m) & (iot < 4 * (m + 1)) for m in range(4)]

    def coarsen16(loads):
        out = None
        for m in range(4):
            t = _perm(loads[m], perms[m])
            out = t if out is None else jnp.where(masks[m], t, out)
        return out

    def group_e(load, avec, bvec):
        g = [[jnp.exp(load(t, d)) for d in range(4)] for t in range(4)]
        e = []
        for d in range(4):
            ssum = g[0][d] + g[1][d] + g[2][d] + g[3][d]
            bos = bvec / ssum
            e.append([g[t][d] * bos + avec for t in range(4)])
        return e

    def x_copy(c, b, q, inbound):
        base = b * _N + c * CH
        if inbound:
            return pltpu.make_async_copy(
                x_hbm.at[pl.ds(base, CH)],
                xbuf_v.at[pl.ds(q * CH, CH)], sem_x)
        return pltpu.make_async_copy(
            xbuf_v.at[pl.ds(q * CH, CH)],
            out_hbm.at[pl.ds(base, CH)], sem_o)

    def gathers(c, q, which=(0, 1, 2)):
        mk = [
            lambda: pltpu.make_async_copy(tab0_sh.at[b0_v.at[c]],
                                          rows0_v.at[pl.ds(q * CH, CH)],
                                          sem_g),
            lambda: pltpu.make_async_copy(tab1_sh.at[b1_v.at[c]],
                                          rows1_v.at[pl.ds(q * 32, 32)],
                                          sem_g),
            lambda: pltpu.make_async_copy(
                tab2_sh.at[b2_v.at[c // 2, pl.ds((c % 2) * 8, 8)]],
                rows2_v.at[pl.ds(q * 8, 8)], sem_g),
        ]
        return [mk[i]() for i in which]

    def batch_body(bi, carry):
        b = wid * BPW + bi
        pltpu.sync_copy(pidx_hbm.at[b], b0_v)
        x_copy(0, b, 0, True).start()
        x_copy(1, b, 1, True).start()

        def c1body(blk, c_):
            loads = [b0_v[blk // 2, pl.ds((blk % 2) * 64 + m * 16, 16)]
                     for m in range(4)]
            b1_v[blk // 2, pl.ds((blk % 2) * 16, 16)] = coarsen16(loads)
            return c_
        lax.fori_loop(0, 64, c1body, 0)

        def c2body(blk, c_):
            loads = [b1_v[blk * 2 + m // 2, pl.ds((m % 2) * 16, 16)]
                     for m in range(4)]
            b2_v[blk, pl.ds(0, 16)] = coarsen16(loads)
            return c_
        lax.fori_loop(0, 16, c2body, 0)

        def bn2(it, c_):
            b2_v[it, pl.ds(0, 16)] = _bins_from_packed(b2_v[it, pl.ds(0, 16)])
            return c_
        lax.fori_loop(0, 16, bn2, 0)

        def bn1(it, c_):
            sl = (it // 2, pl.ds((it % 2) * 16, 16))
            b1_v[sl] = _bins_from_packed(b1_v[sl])
            return c_
        lax.fori_loop(0, 64, bn1, 0)
        for d in gathers(0, 0, which=(1, 2)):
            d.start()

        def bn0(it, c_):
            sl = (it // 8, pl.ds((it % 8) * 16, 16))
            b0_v[sl] = _bins_from_packed(b0_v[sl])
            return c_
        lax.fori_loop(0, 256, bn0, 0)

        gathers(0, 0, which=(0,))[0].start()

        def chunk_body(c, c_):
            q = c % 2

            @pl.when(c + 1 < NCHUNK)
            def _():
                for d in gathers(c + 1, 1 - q):
                    d.start()

            @pl.when(c > 0)
            def _():
                x_copy(c - 1, b, 1 - q, False).wait()

            @pl.when((c > 0) & (c + 1 < NCHUNK))
            def _():
                x_copy(c + 1, b, 1 - q, True).start()

            for d in gathers(c, q):
                d.wait()
            x_copy(c, b, q, True).wait()

            def l2blk(kk2, cc_):
                e2 = group_e(
                    lambda t, d: rows2_v[q * 8 + 4 * kk2 + t,
                                         pl.ds(16 * d, 16)], a2, b2)
                for t2 in range(4):
                    kk1 = 4 * kk2 + t2
                    e1 = group_e(
                        lambda t, d: rows1_v[q * 32 + 4 * kk1 + t,
                                             pl.ds(16 * d, 16)], a1, b1)
                    pr = [[e1[d][t1] * e2[d][t2] for t1 in range(4)]
                          for d in range(4)]
                    for t1 in range(4):
                        kk = 4 * kk1 + t1
                        e0 = group_e(
                            lambda t, d: rows0_v[q * CH + 4 * kk + t,
                                                 pl.ds(16 * d, 16)], a0, b0)
                        for t0 in range(4):
                            row = q * CH + 4 * kk + t0
                            for d in range(4):
                                xv = xbuf_v[row, pl.ds(16 * d, 16)]
                                xbuf_v[row, pl.ds(16 * d, 16)] = \
                                    e0[d][t0] * pr[d][t1] * xv
                return cc_
            lax.fori_loop(0, 2, l2blk, 0)

            x_copy(c, b, q, False).start()
            return c_
        lax.fori_loop(0, NCHUNK, chunk_body, 0)
        x_copy(NCHUNK - 1, b, (NCHUNK - 1) % 2, False).wait()
        return carry
    lax.fori_loop(0, BPW, batch_body, 0)


def _pad_rows(tab):
    return jnp.pad(tab, ((0, _TROWS - tab.shape[0]), (0, 64)))


def kernel(x_level, indices_layer, seq_level, table_0, table_1, table_2):
    B, N, F = x_level.shape
    assert (B, N, F) == (_B, _N, _F)
    mk = (jnp.arange(3) < seq_level).astype(jnp.float32)
    scal = jnp.stack([1.0 - mk[0], mk[0], 1.0 - mk[1], mk[1],
                      1.0 - mk[2], mk[2]])
    eff = scal[:, None] * jnp.ones((1, 16), jnp.float32)

    pidx = (indices_layer[..., 0] |
            (indices_layer[..., 1] << 16)).reshape(B, NCHUNK, CH)

    mesh = plsc.VectorSubcoreMesh(core_axis_name="c", subcore_axis_name="s")
    run = pl.kernel(
        _sc_body, mesh=mesh,
        out_type=jax.ShapeDtypeStruct((B * N, F), jnp.float32),
        scratch_types=[
            pltpu.VMEM_SHARED((_TROWS, 128), jnp.float32),
            pltpu.VMEM_SHARED((_TROWS, 128), jnp.float32),
            pltpu.VMEM_SHARED((_TROWS, 128), jnp.float32),
            pltpu.VMEM((NCHUNK, CH), jnp.int32),
            pltpu.VMEM((32, 32), jnp.int32),
            pltpu.VMEM((16, 16), jnp.int32),
            pltpu.VMEM((2 * CH, 128), jnp.float32),
            pltpu.VMEM((2 * 32, 128), jnp.float32),
            pltpu.VMEM((2 * 8, 128), jnp.float32),
            pltpu.VMEM((2 * CH, EDIM), jnp.float32),
            pltpu.VMEM((6, 16), jnp.float32),
            pltpu.SemaphoreType.DMA,
            pltpu.SemaphoreType.DMA,
            pltpu.SemaphoreType.DMA,
        ])
    out = run(x_level.reshape(B * N, F), pidx,
              _pad_rows(table_0), _pad_rows(table_1), _pad_rows(table_2), eff)
    return out.reshape(B, N, F)
